# baseline (speedup 1.0000x reference)
import functools

import jax
import jax.numpy as jnp
import numpy as np
from jax.experimental import pallas as pl

N = 50000


def _twirls_layer(x, src, dst, conv, n):
    h = jax.nn.relu(x @ conv["lin1"]["w"] + conv["lin1"]["b"])
    y0 = h @ conv["lin2"]["w"] + conv["lin2"]["b"]
    diff = y0[src] - y0[dst]
    nrm = jnp.sqrt(jnp.sum(diff * diff, axis=1) + 1e-12)
    w = 1.0 / jnp.maximum(nrm, 0.2)
    deg_dst = jnp.zeros((n,), jnp.float32).at[dst].add(w) + 1.0
    deg_src = jnp.zeros((n,), jnp.float32).at[src].add(w) + 1.0
    coef = w / jnp.sqrt(deg_src[src] * deg_dst[dst])
    y = y0
    for _ in range(7):
        agg = jnp.zeros_like(y).at[dst].add(coef[:, None] * y[src])
        agg = agg + y / deg_dst[:, None]
        y = 0.5 * (agg + y0)
    return y


def _pool(feat, lin):
    gate = jax.nn.softmax((feat @ lin["w"] + lin["b"]).squeeze(-1), axis=0)
    return jnp.sum(gate[:, None] * feat, axis=0)


def _lstm_dir(xs, p, reverse):
    H = 31

    def step(carry, x):
        h, c = carry
        z = x @ p["Wih"].T + p["bih"] + h @ p["Whh"].T + p["bhh"]
        i, f, g, o = jnp.split(z, 4)
        i = jax.nn.sigmoid(i)
        f = jax.nn.sigmoid(f)
        g = jnp.tanh(g)
        o = jax.nn.sigmoid(o)
        c = f * c + i * g
        h = o * jnp.tanh(c)
        return (h, c), h

    if reverse:
        xs = xs[::-1]
    init = (jnp.zeros((H,), jnp.float32), jnp.zeros((H,), jnp.float32))
    _, hs = jax.lax.scan(step, init, xs)
    if reverse:
        hs = hs[::-1]
    return hs


def _fc_block(flat_ref, w1_ref, b1_ref, w2_ref, out_ref):
    j = pl.program_id(0)
    h = jnp.maximum(flat_ref[...] @ w1_ref[...] + b1_ref[...], 0.0)
    part = h @ w2_ref[...]

    @pl.when(j == 0)
    def _():
        out_ref[...] = jnp.zeros_like(out_ref)

    out_ref[...] += part


def _fc_head(flat, p_in, p_out):
    K, M = p_in["w"].shape
    MP = ((M + 255) // 256) * 256
    w1 = jnp.pad(p_in["w"], ((0, 0), (0, MP - M)))
    b1 = jnp.pad(p_in["b"], (0, MP - M))
    w2 = jnp.pad(p_out["w"], ((0, MP - M), (0, 0)))
    BM = 256
    grid = (MP // BM,)
    acc = pl.pallas_call(
        _fc_block,
        grid=grid,
        in_specs=[
            pl.BlockSpec((1, K), lambda j: (0, 0)),
            pl.BlockSpec((K, BM), lambda j: (0, j)),
            pl.BlockSpec((1, BM), lambda j: (0, j)),
            pl.BlockSpec((BM, 1), lambda j: (j, 0)),
        ],
        out_specs=pl.BlockSpec((1, 1), lambda j: (0, 0)),
        out_shape=jax.ShapeDtypeStruct((1, 1), jnp.float32),
    )(flat, w1, b1[None, :], w2)
    return jax.nn.sigmoid(acc + p_out["b"])


def kernel(protein_x, ligand_x, params, protein_edge_index, ligand_edge_index):
    l_src = ligand_edge_index[0]
    l_dst = ligand_edge_index[1]
    feat_l = ligand_x
    for conv in params["convs"]:
        feat_l = jax.nn.relu(_twirls_layer(feat_l, l_src, l_dst, conv, N))
    protein_rep = _pool(protein_x, params["pool_protein"])
    ligand_rep = _pool(feat_l, params["pool_ligand"])
    seq = jnp.stack([ligand_rep, protein_rep], axis=0)
    seq = jnp.pad(seq, ((0, 138), (0, 0)))
    m = jnp.eye(140, dtype=jnp.float32)
    m = m.at[2:, :].set(0.0).at[:, 2:].set(0.0)
    m = m.at[:, 1].set(1.0).at[1, :].set(1.0).at[1, 1].set(0.0)
    lp = params["lstm"]
    h0 = jnp.concatenate(
        [_lstm_dir(seq, lp["l0_f"], False), _lstm_dir(seq, lp["l0_b"], True)], axis=1)
    h1 = jnp.concatenate(
        [_lstm_dir(h0, lp["l1_f"], False), _lstm_dir(h0, lp["l1_b"], True)], axis=1)
    ap = params["attn"]
    q = (h1 @ ap["wq"]["w"] + ap["wq"]["b"]).reshape(140, 2, 31).transpose(1, 0, 2)
    k = (h1 @ ap["wk"]["w"] + ap["wk"]["b"]).reshape(140, 2, 31).transpose(1, 0, 2)
    v = (h1 @ ap["wv"]["w"] + ap["wv"]["b"]).reshape(140, 2, 31).transpose(1, 0, 2)
    scores = jnp.einsum("htd,hsd->hts", q, k) / np.sqrt(31.0)
    scores = jnp.where(m[None, :, :] == 0.0, -1e9, scores)
    attn = jax.nn.softmax(scores, axis=-1)
    ctx = jnp.einsum("hts,hsd->htd", attn, v).transpose(1, 0, 2).reshape(140, 62)
    out = ctx @ ap["wo"]["w"] + ap["wo"]["b"]
    flat = out.reshape(1, 140 * 62)
    return _fc_head(flat, params["fc_in"], params["fc_out"])


# speedup vs baseline: 1.4864x; 1.4864x over previous
import jax
import jax.numpy as jnp
import numpy as np
from jax import lax
from jax.experimental import pallas as pl
from jax.experimental.pallas import tpu as pltpu
from jax.experimental.pallas import tpu_sc as plsc

N = 50000
N_E = 800000
NP = 50176
F = 32
ECH = 128
NWORK = 32
CPW = 196
EPAD = NWORK * CPW * ECH
SROWS = NP // 16
ZROWS = 196


def _prop_body(y_hbm, src_hbm, dst_hbm, coef_hbm, out_hbm,
               agg_sh, src_v, dst_v, coef_v, rows_v, zbuf):
    c = lax.axis_index("c")
    s = lax.axis_index("s")
    wid = s * 2 + c
    lanes = lax.iota(jnp.int32, 16)

    def zrow(r, _):
        zbuf[r, pl.ds(0, 16)] = jnp.zeros((16,), jnp.float32)
        zbuf[r, pl.ds(16, 16)] = jnp.zeros((16,), jnp.float32)
        return 0
    lax.fori_loop(0, ZROWS, zrow, 0)

    def zcopy(j, _):
        pltpu.sync_copy(zbuf, agg_sh.at[pl.ds(s * SROWS + j * ZROWS, ZROWS)])
        return 0
    lax.fori_loop(0, SROWS // ZROWS, zcopy, 0)
    plsc.subcore_barrier()

    def chunk(ci, _):
        base = (wid * CPW + ci) * ECH
        pltpu.sync_copy(src_hbm.at[pl.ds(base, ECH)], src_v)
        pltpu.sync_copy(dst_hbm.at[pl.ds(base, ECH)], dst_v)
        pltpu.sync_copy(coef_hbm.at[pl.ds(base, ECH)], coef_v)
        pltpu.sync_copy(y_hbm.at[src_v], rows_v)

        def scale(gi, _):
            cf = coef_v[pl.ds(gi * 16, 16)]
            ridx = gi * 16 + lanes
            for f in range(F):
                fidx = jnp.full((16,), f, jnp.int32)
                a = plsc.load_gather(rows_v, [ridx, fidx])
                plsc.store_scatter(rows_v, [ridx, fidx], a * cf)
            return 0
        lax.fori_loop(0, ECH // 16, scale, 0)

        pltpu.sync_copy(rows_v, agg_sh.at[dst_v], add=True)
        return 0
    lax.fori_loop(0, CPW, chunk, 0)
    plsc.subcore_barrier()

    pltpu.sync_copy(agg_sh.at[pl.ds(s * SROWS, SROWS)],
                    out_hbm.at[pl.ds(c * NP + s * SROWS, SROWS)])


def _prop_step(y, srcp, dstp, coefp):
    mesh = plsc.VectorSubcoreMesh(core_axis_name="c", subcore_axis_name="s")
    kfn = pl.kernel(
        _prop_body,
        mesh=mesh,
        compiler_params=pltpu.CompilerParams(needs_layout_passes=False,
                                             use_tc_tiling_on_sc=False),
        out_type=jax.ShapeDtypeStruct((2 * NP, F), jnp.float32),
        scratch_types=[
            pltpu.VMEM_SHARED((NP, F), jnp.float32),
            pltpu.VMEM((ECH,), jnp.int32),
            pltpu.VMEM((ECH,), jnp.int32),
            pltpu.VMEM((ECH,), jnp.float32),
            pltpu.VMEM((ECH, F), jnp.float32),
            pltpu.VMEM((ZROWS, F), jnp.float32),
        ],
    )
    return kfn(y, srcp, dstp, coefp)


def _twirls_layer(x, src, dst, srcp, dstp, conv):
    h = jax.nn.relu(x @ conv["lin1"]["w"] + conv["lin1"]["b"])
    y0 = h @ conv["lin2"]["w"] + conv["lin2"]["b"]
    diff = y0[src] - y0[dst]
    nrm = jnp.sqrt(jnp.sum(diff * diff, axis=1) + 1e-12)
    w = 1.0 / jnp.maximum(nrm, 0.2)
    deg_dst = jnp.zeros((N,), jnp.float32).at[dst].add(w) + 1.0
    deg_src = jnp.zeros((N,), jnp.float32).at[src].add(w) + 1.0
    coef = w / jnp.sqrt(deg_src[src] * deg_dst[dst])
    coefp = jnp.zeros((EPAD,), jnp.float32).at[:N_E].set(coef)

    y0p = jnp.zeros((NP, F), jnp.float32).at[:N, :31].set(y0)
    invd = jnp.ones((NP,), jnp.float32).at[:N].set(1.0 / deg_dst)

    def step(_, y):
        agg2 = _prop_step(y, srcp, dstp, coefp)
        agg = agg2[:NP] + agg2[NP:]
        return 0.5 * (agg + y * invd[:, None] + y0p)

    y = lax.fori_loop(0, 7, step, y0p)
    return jax.nn.relu(y[:N, :31])


def _pool(feat, lin):
    gate = jax.nn.softmax((feat @ lin["w"] + lin["b"]).squeeze(-1), axis=0)
    return jnp.sum(gate[:, None] * feat, axis=0)


def _lstm_dir(xs, p, reverse):
    H = 31

    def step(carry, x):
        h, c = carry
        z = x @ p["Wih"].T + p["bih"] + h @ p["Whh"].T + p["bhh"]
        i, f, g, o = jnp.split(z, 4)
        i = jax.nn.sigmoid(i)
        f = jax.nn.sigmoid(f)
        g = jnp.tanh(g)
        o = jax.nn.sigmoid(o)
        c = f * c + i * g
        h = o * jnp.tanh(c)
        return (h, c), h

    if reverse:
        xs = xs[::-1]
    init = (jnp.zeros((H,), jnp.float32), jnp.zeros((H,), jnp.float32))
    _, hs = jax.lax.scan(step, init, xs)
    if reverse:
        hs = hs[::-1]
    return hs


def _fc_block(flat_ref, w1_ref, b1_ref, w2_ref, out_ref):
    j = pl.program_id(0)
    h = jnp.maximum(flat_ref[...] @ w1_ref[...] + b1_ref[...], 0.0)
    part = h @ w2_ref[...]

    @pl.when(j == 0)
    def _():
        out_ref[...] = jnp.zeros_like(out_ref)

    out_ref[...] += part


def _fc_head(flat, p_in, p_out):
    K, M = p_in["w"].shape
    MP = ((M + 255) // 256) * 256
    w1 = jnp.pad(p_in["w"], ((0, 0), (0, MP - M)))
    b1 = jnp.pad(p_in["b"], (0, MP - M))
    w2 = jnp.pad(p_out["w"], ((0, MP - M), (0, 0)))
    BM = 256
    acc = pl.pallas_call(
        _fc_block,
        grid=(MP // BM,),
        in_specs=[
            pl.BlockSpec((1, K), lambda j: (0, 0)),
            pl.BlockSpec((K, BM), lambda j: (0, j)),
            pl.BlockSpec((1, BM), lambda j: (0, j)),
            pl.BlockSpec((BM, 1), lambda j: (j, 0)),
        ],
        out_specs=pl.BlockSpec((1, 1), lambda j: (0, 0)),
        out_shape=jax.ShapeDtypeStruct((1, 1), jnp.float32),
    )(flat, w1, b1[None, :], w2)
    return jax.nn.sigmoid(acc + p_out["b"])


def kernel(protein_x, ligand_x, params, protein_edge_index, ligand_edge_index):
    l_src = ligand_edge_index[0]
    l_dst = ligand_edge_index[1]
    srcp = jnp.zeros((EPAD,), jnp.int32).at[:N_E].set(l_src)
    dstp = jnp.zeros((EPAD,), jnp.int32).at[:N_E].set(l_dst)

    feat_l = ligand_x
    for conv in params["convs"]:
        feat_l = _twirls_layer(feat_l, l_src, l_dst, srcp, dstp, conv)

    protein_rep = _pool(protein_x, params["pool_protein"])
    ligand_rep = _pool(feat_l, params["pool_ligand"])
    seq = jnp.stack([ligand_rep, protein_rep], axis=0)
    seq = jnp.pad(seq, ((0, 138), (0, 0)))
    m = jnp.eye(140, dtype=jnp.float32)
    m = m.at[2:, :].set(0.0).at[:, 2:].set(0.0)
    m = m.at[:, 1].set(1.0).at[1, :].set(1.0).at[1, 1].set(0.0)
    lp = params["lstm"]
    h0 = jnp.concatenate(
        [_lstm_dir(seq, lp["l0_f"], False), _lstm_dir(seq, lp["l0_b"], True)],
        axis=1)
    h1 = jnp.concatenate(
        [_lstm_dir(h0, lp["l1_f"], False), _lstm_dir(h0, lp["l1_b"], True)],
        axis=1)
    ap = params["attn"]
    q = (h1 @ ap["wq"]["w"] + ap["wq"]["b"]).reshape(140, 2, 31).transpose(1, 0, 2)
    k = (h1 @ ap["wk"]["w"] + ap["wk"]["b"]).reshape(140, 2, 31).transpose(1, 0, 2)
    v = (h1 @ ap["wv"]["w"] + ap["wv"]["b"]).reshape(140, 2, 31).transpose(1, 0, 2)
    scores = jnp.einsum("htd,hsd->hts", q, k) / np.sqrt(31.0)
    scores = jnp.where(m[None, :, :] == 0.0, -1e9, scores)
    attn = jax.nn.softmax(scores, axis=-1)
    ctx = jnp.einsum("hts,hsd->htd", attn, v).transpose(1, 0, 2).reshape(140, 62)
    out = ctx @ ap["wo"]["w"] + ap["wo"]["b"]
    flat = out.reshape(1, 140 * 62)
    return _fc_head(flat, params["fc_in"], params["fc_out"])


# speedup vs baseline: 1.7002x; 1.1439x over previous
import jax
import jax.numpy as jnp
import numpy as np
from jax import lax
from jax.experimental import pallas as pl
from jax.experimental.pallas import tpu as pltpu
from jax.experimental.pallas import tpu_sc as plsc

N = 50000
N_E = 800000
NP = 50176
F = 32
ECH = 128
NWORK = 32
CPW = 196
BLK = 14
NBLK = CPW // BLK
EPAD = NWORK * CPW * ECH
NCH = EPAD // ECH
SROWS = NP // 16
ZROWS = 196


def _prop_body(y_hbm, src_hbm, dst_hbm, coef_hbm, out_hbm,
               agg_sh, srcb_v, dstb_v, coefb_v, rows_v, zbuf, sem):
    c = lax.axis_index("c")
    s = lax.axis_index("s")
    wid = s * 2 + c
    lanes = lax.iota(jnp.int32, 16)

    def zrow(r, _):
        zbuf[r, pl.ds(0, 16)] = jnp.zeros((16,), jnp.float32)
        zbuf[r, pl.ds(16, 16)] = jnp.zeros((16,), jnp.float32)
        return 0
    lax.fori_loop(0, ZROWS, zrow, 0)

    def zcopy(j, _):
        pltpu.sync_copy(zbuf, agg_sh.at[pl.ds(s * SROWS + j * ZROWS, ZROWS)])
        return 0
    lax.fori_loop(0, SROWS // ZROWS, zcopy, 0)
    plsc.subcore_barrier()

    def process(j, buf):
        pltpu.make_async_copy(y_hbm.at[srcb_v.at[j]],
                              rows_v.at[buf], sem).wait()
        pidx = jnp.full((16,), buf, jnp.int32)

        def scale(gi, _):
            cf = coefb_v[j, pl.ds(gi * 16, 16)]
            ridx = gi * 16 + lanes
            for f in range(F):
                fidx = jnp.full((16,), f, jnp.int32)
                a = plsc.load_gather(rows_v, [pidx, ridx, fidx])
                plsc.store_scatter(rows_v, [pidx, ridx, fidx], a * cf)
            return 0
        lax.fori_loop(0, ECH // 16, scale, 0)

        pltpu.sync_copy(rows_v.at[buf], agg_sh.at[dstb_v.at[j]], add=True)

    def blockf(bi, _):
        row0 = (wid * NBLK + bi) * BLK
        pltpu.sync_copy(src_hbm.at[pl.ds(row0, BLK)], srcb_v)
        pltpu.sync_copy(dst_hbm.at[pl.ds(row0, BLK)], dstb_v)
        pltpu.sync_copy(coef_hbm.at[pl.ds(row0, BLK)], coefb_v)
        pltpu.async_copy(y_hbm.at[srcb_v.at[0]], rows_v.at[0], sem)

        def pair(jj, _):
            j0 = 2 * jj
            pltpu.async_copy(y_hbm.at[srcb_v.at[j0 + 1]], rows_v.at[1], sem)
            process(j0, 0)

            @pl.when(j0 + 2 < BLK)
            def _():
                pltpu.async_copy(y_hbm.at[srcb_v.at[j0 + 2]],
                                 rows_v.at[0], sem)
            process(j0 + 1, 1)
            return 0
        lax.fori_loop(0, BLK // 2, pair, 0)
        return 0
    lax.fori_loop(0, NBLK, blockf, 0)
    plsc.subcore_barrier()

    pltpu.sync_copy(agg_sh.at[pl.ds(s * SROWS, SROWS)],
                    out_hbm.at[pl.ds(c * NP + s * SROWS, SROWS)])


def _prop_step(y, srcp, dstp, coefp):
    mesh = plsc.VectorSubcoreMesh(core_axis_name="c", subcore_axis_name="s")
    kfn = pl.kernel(
        _prop_body,
        mesh=mesh,
        compiler_params=pltpu.CompilerParams(needs_layout_passes=False,
                                             use_tc_tiling_on_sc=False),
        out_type=jax.ShapeDtypeStruct((2 * NP, F), jnp.float32),
        scratch_types=[
            pltpu.VMEM_SHARED((NP, F), jnp.float32),
            pltpu.VMEM((BLK, ECH), jnp.int32),
            pltpu.VMEM((BLK, ECH), jnp.int32),
            pltpu.VMEM((BLK, ECH), jnp.float32),
            pltpu.VMEM((2, ECH, F), jnp.float32),
            pltpu.VMEM((ZROWS, F), jnp.float32),
            pltpu.SemaphoreType.DMA,
        ],
    )
    return kfn(y, srcp, dstp, coefp)


def _twirls_layer(x, src, dst, srcp, dstp, conv):
    h = jax.nn.relu(x @ conv["lin1"]["w"] + conv["lin1"]["b"])
    y0 = h @ conv["lin2"]["w"] + conv["lin2"]["b"]
    diff = y0[src] - y0[dst]
    nrm = jnp.sqrt(jnp.sum(diff * diff, axis=1) + 1e-12)
    w = 1.0 / jnp.maximum(nrm, 0.2)
    deg_dst = jnp.zeros((N,), jnp.float32).at[dst].add(w) + 1.0
    deg_src = jnp.zeros((N,), jnp.float32).at[src].add(w) + 1.0
    coef = w / jnp.sqrt(deg_src[src] * deg_dst[dst])
    coefp = jnp.zeros((EPAD,), jnp.float32).at[:N_E].set(coef)
    coefp = coefp.reshape(NCH, ECH)

    y0p = jnp.zeros((NP, F), jnp.float32).at[:N, :31].set(y0)
    invd = jnp.ones((NP,), jnp.float32).at[:N].set(1.0 / deg_dst)

    def step(_, y):
        agg2 = _prop_step(y, srcp, dstp, coefp)
        agg = agg2[:NP] + agg2[NP:]
        return 0.5 * (agg + y * invd[:, None] + y0p)

    y = lax.fori_loop(0, 7, step, y0p)
    return jax.nn.relu(y[:N, :31])


def _pool(feat, lin):
    gate = jax.nn.softmax((feat @ lin["w"] + lin["b"]).squeeze(-1), axis=0)
    return jnp.sum(gate[:, None] * feat, axis=0)


def _lstm_dir(xs, p, reverse):
    H = 31

    def step(carry, x):
        h, c = carry
        z = x @ p["Wih"].T + p["bih"] + h @ p["Whh"].T + p["bhh"]
        i, f, g, o = jnp.split(z, 4)
        i = jax.nn.sigmoid(i)
        f = jax.nn.sigmoid(f)
        g = jnp.tanh(g)
        o = jax.nn.sigmoid(o)
        c = f * c + i * g
        h = o * jnp.tanh(c)
        return (h, c), h

    if reverse:
        xs = xs[::-1]
    init = (jnp.zeros((H,), jnp.float32), jnp.zeros((H,), jnp.float32))
    _, hs = jax.lax.scan(step, init, xs)
    if reverse:
        hs = hs[::-1]
    return hs


def _fc_block(flat_ref, w1_ref, b1_ref, w2_ref, out_ref):
    j = pl.program_id(0)
    h = jnp.maximum(flat_ref[...] @ w1_ref[...] + b1_ref[...], 0.0)
    part = h @ w2_ref[...]

    @pl.when(j == 0)
    def _():
        out_ref[...] = jnp.zeros_like(out_ref)

    out_ref[...] += part


def _fc_head(flat, p_in, p_out):
    K, M = p_in["w"].shape
    MP = ((M + 255) // 256) * 256
    w1 = jnp.pad(p_in["w"], ((0, 0), (0, MP - M)))
    b1 = jnp.pad(p_in["b"], (0, MP - M))
    w2 = jnp.pad(p_out["w"], ((0, MP - M), (0, 0)))
    BM = 256
    acc = pl.pallas_call(
        _fc_block,
        grid=(MP // BM,),
        in_specs=[
            pl.BlockSpec((1, K), lambda j: (0, 0)),
            pl.BlockSpec((K, BM), lambda j: (0, j)),
            pl.BlockSpec((1, BM), lambda j: (0, j)),
            pl.BlockSpec((BM, 1), lambda j: (j, 0)),
        ],
        out_specs=pl.BlockSpec((1, 1), lambda j: (0, 0)),
        out_shape=jax.ShapeDtypeStruct((1, 1), jnp.float32),
    )(flat, w1, b1[None, :], w2)
    return jax.nn.sigmoid(acc + p_out["b"])


def kernel(protein_x, ligand_x, params, protein_edge_index, ligand_edge_index):
    l_src = ligand_edge_index[0]
    l_dst = ligand_edge_index[1]
    srcp = jnp.zeros((EPAD,), jnp.int32).at[:N_E].set(l_src).reshape(NCH, ECH)
    dstp = jnp.zeros((EPAD,), jnp.int32).at[:N_E].set(l_dst).reshape(NCH, ECH)

    feat_l = ligand_x
    for conv in params["convs"]:
        feat_l = _twirls_layer(feat_l, l_src, l_dst, srcp, dstp, conv)

    protein_rep = _pool(protein_x, params["pool_protein"])
    ligand_rep = _pool(feat_l, params["pool_ligand"])
    seq = jnp.stack([ligand_rep, protein_rep], axis=0)
    seq = jnp.pad(seq, ((0, 138), (0, 0)))
    m = jnp.eye(140, dtype=jnp.float32)
    m = m.at[2:, :].set(0.0).at[:, 2:].set(0.0)
    m = m.at[:, 1].set(1.0).at[1, :].set(1.0).at[1, 1].set(0.0)
    lp = params["lstm"]
    h0 = jnp.concatenate(
        [_lstm_dir(seq, lp["l0_f"], False), _lstm_dir(seq, lp["l0_b"], True)],
        axis=1)
    h1 = jnp.concatenate(
        [_lstm_dir(h0, lp["l1_f"], False), _lstm_dir(h0, lp["l1_b"], True)],
        axis=1)
    ap = params["attn"]
    q = (h1 @ ap["wq"]["w"] + ap["wq"]["b"]).reshape(140, 2, 31).transpose(1, 0, 2)
    k = (h1 @ ap["wk"]["w"] + ap["wk"]["b"]).reshape(140, 2, 31).transpose(1, 0, 2)
    v = (h1 @ ap["wv"]["w"] + ap["wv"]["b"]).reshape(140, 2, 31).transpose(1, 0, 2)
    scores = jnp.einsum("htd,hsd->hts", q, k) / np.sqrt(31.0)
    scores = jnp.where(m[None, :, :] == 0.0, -1e9, scores)
    attn = jax.nn.softmax(scores, axis=-1)
    ctx = jnp.einsum("hts,hsd->htd", attn, v).transpose(1, 0, 2).reshape(140, 62)
    out = ctx @ ap["wo"]["w"] + ap["wo"]["b"]
    flat = out.reshape(1, 140 * 62)
    return _fc_head(flat, params["fc_in"], params["fc_out"])
